# baseline (speedup 1.0000x reference)
import functools

import jax
import jax.numpy as jnp
from jax import lax
from jax.experimental import pallas as pl
from jax.experimental.pallas import tpu as pltpu
from jax.experimental.pallas import tpu_sc as plsc

N = 16384
S1 = 512
S2 = 16
PAD = 64
EMB = 384

TILE = 1024
NT = N // TILE

SEG_PER_W = 16
CH = 64

OUT_ROWS = S2 * PAD + 8

_NINF = float("-inf")


def _k1_body(x_ref, sp_ref, w1_ref, b1_ref, w2_ref, b2_ref,
             wp1_ref, bp1_ref, wp2_ref, bp2_ref, h_ref, pos_ref):
    t = pl.program_id(0)
    h = jnp.dot(x_ref[...], w1_ref[...], preferred_element_type=jnp.float32)
    h = h + b1_ref[...]
    h = jnp.dot(h, w2_ref[...], preferred_element_type=jnp.float32)
    h = h + b2_ref[...]
    h_ref[...] = h

    @pl.when(t == 0)
    def _():
        p = jnp.dot(sp_ref[...], wp1_ref[...],
                    preferred_element_type=jnp.float32)
        p = p + bp1_ref[...]
        p = 0.5 * p * (1.0 + jax.lax.erf(p * 0.7071067811865476))
        p = jnp.dot(p, wp2_ref[...], preferred_element_type=jnp.float32)
        pos_ref[...] = p + bp2_ref[...]


def _scal(meta_v, slot):
    return meta_v[pl.ds(slot * 16, 16)][0].astype(jnp.int32)


def _sc_reduce(src_hbm, meta_v, bufs, sems, tail, out_loc, ngrp):
    c = ngrp * 16
    starts = [_scal(meta_v, i) for i in range(SEG_PER_W)]
    cnts = [_scal(meta_v, SEG_PER_W + i) for i in range(SEG_PER_W)]
    start8s = [(s // 8) * 8 for s in starts]
    bases = [jnp.minimum(s8, N - CH) for s8 in start8s]

    handle = pltpu.async_copy(src_hbm.at[pl.ds(bases[0], CH)], bufs[0],
                              sems[0])
    for i in range(SEG_PER_W):
        start, cnt, start8 = starts[i], cnts[i], start8s[i]
        nch = (start - start8 + cnt + CH - 1) // CH
        handle.wait()
        if i + 1 < SEG_PER_W:
            handle = pltpu.async_copy(
                src_hbm.at[pl.ds(bases[i + 1], CH)], bufs[(i + 1) % 2],
                sems[(i + 1) % 2])

        def red(buf, cbase, acc, start=start, cnt=cnt):
            lo = jnp.maximum(start, cbase) - cbase
            hi = jnp.minimum(start + cnt, cbase + CH) - cbase
            unroll = 4 if ngrp <= 16 else 2
            niter = (hi - lo + unroll - 1) // unroll

            def row_loop(q, acc2):
                out = acc2
                for j in range(unroll):
                    r = jnp.minimum(lo + q * unroll + j, hi - 1)
                    out = [jnp.maximum(out[g], buf[r, pl.ds(g * 16, 16)])
                           for g in range(ngrp)]
                return out

            return lax.fori_loop(0, niter, row_loop, acc)

        acc = [jnp.full((16,), _NINF, jnp.float32) for _ in range(ngrp)]
        acc = red(bufs[i % 2], bases[i], acc)

        def ch_loop(k, acc2, start8=start8, red=red):
            cbase = jnp.minimum(start8 + k * CH, N - CH)
            pltpu.sync_copy(src_hbm.at[pl.ds(cbase, CH)], tail)
            return red(tail, cbase, acc2)

        acc = lax.fori_loop(1, nch, ch_loop, acc)
        for g in range(ngrp):
            out_loc[pl.ds(i * c + g * 16, 16)] = acc[g]


def _sca_body(h_hbm, meta_hbm, g_hbm, meta_v, buf0, buf1, tail, g_loc,
              sem0, sem1):
    wid = lax.axis_index("s") * 2 + lax.axis_index("c")
    pltpu.sync_copy(meta_hbm.at[pl.ds(wid * 32 * 16, 32 * 16)], meta_v)
    _sc_reduce(h_hbm, meta_v, (buf0, buf1), (sem0, sem1), tail, g_loc, 16)
    pltpu.sync_copy(g_loc, g_hbm.at[pl.ds(wid * SEG_PER_W * 256,
                                          SEG_PER_W * 256)])


def _k3_body(h_ref, seg_ref, g_ref, w3a_ref, b3_ref, w3b_ref, w4_ref, b4_ref,
             f2_ref, a_ref):
    t = pl.program_id(0)

    @pl.when(t == 0)
    def _():
        a = jnp.dot(g_ref[...], w3a_ref[...],
                    preferred_element_type=jnp.float32)
        a_ref[...] = (a + b3_ref[...]).astype(jnp.bfloat16)

    seg = seg_ref[...]
    iota = lax.broadcasted_iota(jnp.int32, (TILE, S1), 1)
    onehot = (seg == iota).astype(jnp.bfloat16)
    za = jnp.dot(onehot, a_ref[...], preferred_element_type=jnp.float32)
    h = h_ref[...]
    zb = jnp.dot(h.astype(jnp.bfloat16), w3b_ref[...],
                 preferred_element_type=jnp.float32)
    z = jnp.maximum(za + zb, 0.0)
    f2 = jnp.dot(z.astype(jnp.bfloat16), w4_ref[...],
                 preferred_element_type=jnp.float32)
    f2_ref[...] = f2 + b4_ref[...]


def _scb_body(f2_hbm, pos_hbm, meta_hbm, tok_hbm, pout_hbm,
              meta_v, buf0, buf1, tail, t_loc, pos_loc, zrow,
              sem0, sem1, semm, sem):
    wid = lax.axis_index("s") * 2 + lax.axis_index("c")
    base = wid * SEG_PER_W
    pltpu.sync_copy(meta_hbm.at[pl.ds(wid * 64 * 16, 64 * 16)], meta_v)

    for g in range(EMB // 16):
        zrow[pl.ds(g * 16, 16)] = jnp.zeros((16,), jnp.float32)
    z0 = _scal(meta_v, 48)
    z1 = _scal(meta_v, 49)

    def fire(r, carry):
        pltpu.async_copy(zrow, tok_hbm.at[pl.ds(r * EMB, EMB)], semm)
        pltpu.async_copy(zrow, pout_hbm.at[pl.ds(r * EMB, EMB)], semm)
        return carry

    lax.fori_loop(z0, z1, fire, 0)

    pltpu.sync_copy(pos_hbm.at[pl.ds(base * EMB, SEG_PER_W * EMB)], pos_loc)

    _sc_reduce(f2_hbm, meta_v, (buf0, buf1), (sem0, sem1), tail, t_loc,
               EMB // 16)

    def drain(r, carry):
        pltpu.make_async_copy(meta_hbm.at[pl.ds(0, EMB)], zrow, semm).wait()
        return carry

    lax.fori_loop(0, 2 * (z1 - z0), drain, 0)

    copies = []
    for i in range(SEG_PER_W):
        d = _scal(meta_v, 32 + i)
        copies.append(pltpu.async_copy(
            t_loc.at[pl.ds(i * EMB, EMB)],
            tok_hbm.at[pl.ds(d * EMB, EMB)], sem))
        copies.append(pltpu.async_copy(
            pos_loc.at[pl.ds(i * EMB, EMB)],
            pout_hbm.at[pl.ds(d * EMB, EMB)], sem))
    for c in copies:
        c.wait()


def kernel(full_features, sp_coords, full_super_indices_10,
           full_super_indices_21, W1, b1, W2, b2, W3, b3, W4, b4,
           Wp1, bp1, Wp2, bp2):
    f32 = jnp.float32
    i32 = jnp.int32
    seg10 = full_super_indices_10.astype(i32)
    seg21 = full_super_indices_21.astype(i32)
    seg_col = seg10.reshape(N, 1)

    counts10 = jnp.sum(
        (seg10[:, None] == jnp.arange(S1, dtype=i32)[None, :]).astype(i32),
        axis=0)
    starts10 = (jnp.cumsum(counts10) - counts10).astype(i32)
    counts21 = jnp.sum(
        (seg21[:, None] == jnp.arange(S2, dtype=i32)[None, :]).astype(i32),
        axis=0)
    starts21 = jnp.cumsum(counts21) - counts21
    rank = jnp.arange(S1, dtype=i32) - starts21[seg21].astype(i32)
    valid = rank < PAD
    dd = jnp.where(valid, seg21 * PAD + rank, S2 * PAD).astype(i32)

    dmin = jnp.where(valid, dd, S2 * PAD)
    suffmin = lax.cummin(dmin[::-1])[::-1]
    zone_first = suffmin[:: SEG_PER_W]
    zstart = zone_first.at[0].set(0).astype(i32)
    zend = jnp.concatenate([zstart[1:], jnp.array([S2 * PAD], i32)])

    st32 = starts10.reshape(32, SEG_PER_W)
    ct32 = counts10.reshape(32, SEG_PER_W)
    dd32 = dd.reshape(32, SEG_PER_W)
    meta_a = jnp.concatenate([st32, ct32], axis=1)
    meta_a = jnp.broadcast_to(
        meta_a.astype(f32)[:, :, None], (32, 32, 16)).reshape(-1)
    meta_b = jnp.concatenate(
        [st32, ct32, dd32, zstart[:, None], zend[:, None],
         jnp.zeros((32, 14), i32)], axis=1)
    meta_b = jnp.broadcast_to(
        meta_b.astype(f32)[:, :, None], (32, 64, 16)).reshape(-1)

    w1t = W1.T
    w2t = W2.T
    w3t = W3.T
    w3a = w3t[:256]
    w3b = w3t[256:].astype(jnp.bfloat16)
    w4t = W4.T.astype(jnp.bfloat16)
    wp1t = Wp1.T
    wp2t = Wp2.T
    b1r = b1.reshape(1, -1)
    b2r = b2.reshape(1, -1)
    b3r = b3.reshape(1, -1)
    b4r = b4.reshape(1, -1)
    bp1r = bp1.reshape(1, -1)
    bp2r = bp2.reshape(1, -1)

    full = lambda shape: pl.BlockSpec(shape, lambda t: (0,) * len(shape))
    row_blk = lambda c: pl.BlockSpec((TILE, c), lambda t: (t, 0))

    h, pos = pl.pallas_call(
        _k1_body,
        grid=(NT,),
        in_specs=[row_blk(6), full((S1, 3)), full((6, 128)), full((1, 128)),
                  full((128, 256)), full((1, 256)), full((3, 128)),
                  full((1, 128)), full((128, EMB)), full((1, EMB))],
        out_specs=[row_blk(256), full((S1, EMB))],
        out_shape=[jax.ShapeDtypeStruct((N, 256), f32),
                   jax.ShapeDtypeStruct((S1, EMB), f32)],
    )(full_features, sp_coords, w1t, b1r, w2t, b2r, wp1t, bp1r, wp2t, bp2r)

    mesh = plsc.VectorSubcoreMesh(core_axis_name="c", subcore_axis_name="s")

    sca = functools.partial(
        pl.kernel, mesh=mesh,
        out_type=jax.ShapeDtypeStruct((S1 * 256,), f32),
        scratch_types=[
            pltpu.VMEM((32 * 16,), f32),
            pltpu.VMEM((CH, 256), f32),
            pltpu.VMEM((CH, 256), f32),
            pltpu.VMEM((CH, 256), f32),
            pltpu.VMEM((SEG_PER_W * 256,), f32),
            pltpu.SemaphoreType.DMA,
            pltpu.SemaphoreType.DMA,
        ],
    )(_sca_body)
    G = sca(h, meta_a).reshape(S1, 256)

    f2 = pl.pallas_call(
        _k3_body,
        grid=(NT,),
        in_specs=[row_blk(256), row_blk(1), full((S1, 256)),
                  full((256, 512)), full((1, 512)), full((256, 512)),
                  full((512, EMB)), full((1, EMB))],
        out_specs=row_blk(EMB),
        out_shape=jax.ShapeDtypeStruct((N, EMB), f32),
        scratch_shapes=[pltpu.VMEM((S1, 512), jnp.bfloat16)],
    )(h, seg_col, G, w3a, b3r, w3b, w4t, b4r)

    scb = functools.partial(
        pl.kernel, mesh=mesh,
        out_type=[jax.ShapeDtypeStruct((OUT_ROWS * EMB,), f32),
                  jax.ShapeDtypeStruct((OUT_ROWS * EMB,), f32)],
        scratch_types=[
            pltpu.VMEM((64 * 16,), f32),
            pltpu.VMEM((CH, EMB), f32),
            pltpu.VMEM((CH, EMB), f32),
            pltpu.VMEM((CH, EMB), f32),
            pltpu.VMEM((SEG_PER_W * EMB,), f32),
            pltpu.VMEM((SEG_PER_W * EMB,), f32),
            pltpu.VMEM((EMB,), f32),
            pltpu.SemaphoreType.DMA,
            pltpu.SemaphoreType.DMA,
            pltpu.SemaphoreType.DMA,
            pltpu.SemaphoreType.DMA,
        ],
    )(_scb_body)
    tok_flat, pos_flat = scb(f2, pos.reshape(-1), meta_b)

    tok_p = tok_flat[:S2 * PAD * EMB].reshape(1, S2, PAD, EMB)
    pos_p = pos_flat[:S2 * PAD * EMB].reshape(1, S2, PAD, EMB)
    return (tok_p, pos_p)


# speedup vs baseline: 1.0287x; 1.0287x over previous
import functools

import jax
import jax.numpy as jnp
from jax import lax
from jax.experimental import pallas as pl
from jax.experimental.pallas import tpu as pltpu
from jax.experimental.pallas import tpu_sc as plsc

N = 16384
S1 = 512
S2 = 16
PAD = 64
EMB = 384

TILE = 1024
NT = N // TILE

SEG_PER_W = 16
CH = 64

OUT_ROWS = S2 * PAD + 8

_NINF = float("-inf")


def _k1_body(x_ref, sp_ref, w1_ref, b1_ref, w2_ref, b2_ref,
             wp1_ref, bp1_ref, wp2_ref, bp2_ref, h_ref, pos_ref):
    t = pl.program_id(0)
    h = jnp.dot(x_ref[...], w1_ref[...], preferred_element_type=jnp.float32)
    h = h + b1_ref[...]
    h = jnp.dot(h, w2_ref[...], preferred_element_type=jnp.float32)
    h = h + b2_ref[...]
    h_ref[...] = h

    @pl.when(t == 0)
    def _():
        p = jnp.dot(sp_ref[...], wp1_ref[...],
                    preferred_element_type=jnp.float32)
        p = p + bp1_ref[...]
        p = 0.5 * p * (1.0 + jax.lax.erf(p * 0.7071067811865476))
        p = jnp.dot(p, wp2_ref[...], preferred_element_type=jnp.float32)
        pos_ref[...] = p + bp2_ref[...]


def _scal(meta_v, slot):
    return meta_v[pl.ds(slot * 16, 16)][0].astype(jnp.int32)


def _sc_reduce(src_hbm, meta_v, bufs, sems, tail, out_loc, cols):
    ngrp = cols // 16
    starts = [_scal(meta_v, i) for i in range(SEG_PER_W)]
    cnts = [_scal(meta_v, SEG_PER_W + i) for i in range(SEG_PER_W)]
    start16s = [(s // 16) * 16 for s in starts]
    bases = [jnp.minimum(s16, N - CH) for s16 in start16s]

    handle = pltpu.async_copy(src_hbm.at[pl.ds(bases[0], CH)], bufs[0],
                              sems[0])
    for i in range(SEG_PER_W):
        start, cnt, start16 = starts[i], cnts[i], start16s[i]
        nch = (start - start16 + cnt + CH - 1) // CH
        handle.wait()
        if i + 1 < SEG_PER_W:
            handle = pltpu.async_copy(
                src_hbm.at[pl.ds(bases[i + 1], CH)], bufs[(i + 1) % 2],
                sems[(i + 1) % 2])

        def red(buf, cbase, acc, start=start, cnt=cnt):
            lo = jnp.maximum(start, cbase) - cbase
            hi = jnp.minimum(start + cnt, cbase + CH) - cbase

            def row_loop(r, acc2):
                return [jnp.maximum(acc2[g], buf[r, pl.ds(g * 16, 16)])
                        for g in range(ngrp)]

            return lax.fori_loop(lo, hi, row_loop, acc)

        acc = [jnp.full((16,), _NINF, jnp.float32) for _ in range(ngrp)]
        acc = red(bufs[i % 2], bases[i], acc)

        def ch_loop(k, acc2, start16=start16, red=red):
            cbase = jnp.minimum(start16 + k * CH, N - CH)
            pltpu.sync_copy(src_hbm.at[pl.ds(cbase, CH)], tail)
            return red(tail, cbase, acc2)

        acc = lax.fori_loop(1, nch, ch_loop, acc)
        for g in range(ngrp):
            out_loc[pl.ds(i * cols + g * 16, 16)] = acc[g]


def _sca_body(h_hbm, meta_hbm, g_hbm, meta_v, buf0, buf1, tail, g_loc,
              sem0, sem1):
    wid = lax.axis_index("s") * 2 + lax.axis_index("c")
    pltpu.sync_copy(meta_hbm.at[pl.ds(wid * 32 * 16, 32 * 16)], meta_v)
    _sc_reduce(h_hbm, meta_v, (buf0, buf1), (sem0, sem1), tail, g_loc, 256)
    pltpu.sync_copy(g_loc, g_hbm.at[pl.ds(wid * SEG_PER_W * 256,
                                          SEG_PER_W * 256)])


def _k3_body(h_ref, seg_ref, g_ref, w3a_ref, b3_ref, w3b_ref, w4_ref, b4_ref,
             f2_ref, a_ref):
    t = pl.program_id(0)

    @pl.when(t == 0)
    def _():
        a = jnp.dot(g_ref[...], w3a_ref[...],
                    preferred_element_type=jnp.float32)
        a_ref[...] = (a + b3_ref[...]).astype(jnp.bfloat16)

    seg = seg_ref[...]
    iota = lax.broadcasted_iota(jnp.int32, (TILE, S1), 1)
    onehot = (seg == iota).astype(jnp.bfloat16)
    za = jnp.dot(onehot, a_ref[...], preferred_element_type=jnp.float32)
    zb = jnp.dot(h_ref[...].astype(jnp.bfloat16), w3b_ref[...],
                 preferred_element_type=jnp.float32)
    z = jnp.maximum(za + zb, 0.0)
    f2 = jnp.dot(z.astype(jnp.bfloat16), w4_ref[...],
                 preferred_element_type=jnp.float32)
    f2_ref[...] = f2 + b4_ref[...]


def _scb_body(f2_hbm, pos_hbm, meta_hbm, tok_hbm, pout_hbm,
              meta_v, buf0, buf1, tail, t_loc, pos_loc, zrow,
              sem0, sem1, semm, sem):
    wid = lax.axis_index("s") * 2 + lax.axis_index("c")
    base = wid * SEG_PER_W
    pltpu.sync_copy(meta_hbm.at[pl.ds(wid * 64 * 16, 64 * 16)], meta_v)

    for g in range(EMB // 16):
        zrow[pl.ds(g * 16, 16)] = jnp.zeros((16,), jnp.float32)
    z0 = _scal(meta_v, 48)
    z1 = _scal(meta_v, 49)

    def fire(r, carry):
        pltpu.async_copy(zrow, tok_hbm.at[pl.ds(r * EMB, EMB)], semm)
        pltpu.async_copy(zrow, pout_hbm.at[pl.ds(r * EMB, EMB)], semm)
        return carry

    lax.fori_loop(z0, z1, fire, 0)

    pltpu.sync_copy(pos_hbm.at[pl.ds(base * EMB, SEG_PER_W * EMB)], pos_loc)

    _sc_reduce(f2_hbm, meta_v, (buf0, buf1), (sem0, sem1), tail, t_loc,
               EMB)

    def drain(r, carry):
        pltpu.make_async_copy(pos_hbm.at[pl.ds(0, EMB)], zrow, semm).wait()
        return carry

    lax.fori_loop(0, 2 * (z1 - z0), drain, 0)

    copies = []
    for i in range(SEG_PER_W):
        d = _scal(meta_v, 32 + i)
        copies.append(pltpu.async_copy(
            t_loc.at[pl.ds(i * EMB, EMB)],
            tok_hbm.at[pl.ds(d * EMB, EMB)], sem))
        copies.append(pltpu.async_copy(
            pos_loc.at[pl.ds(i * EMB, EMB)],
            pout_hbm.at[pl.ds(d * EMB, EMB)], sem))
    for c in copies:
        c.wait()


def kernel(full_features, sp_coords, full_super_indices_10,
           full_super_indices_21, W1, b1, W2, b2, W3, b3, W4, b4,
           Wp1, bp1, Wp2, bp2):
    f32 = jnp.float32
    i32 = jnp.int32
    seg10 = full_super_indices_10.astype(i32)
    seg21 = full_super_indices_21.astype(i32)
    seg_col = seg10.reshape(N, 1)

    counts10 = jnp.sum(
        (seg10[:, None] == jnp.arange(S1, dtype=i32)[None, :]).astype(i32),
        axis=0)
    starts10 = (jnp.cumsum(counts10) - counts10).astype(i32)
    counts21 = jnp.sum(
        (seg21[:, None] == jnp.arange(S2, dtype=i32)[None, :]).astype(i32),
        axis=0)
    starts21 = jnp.cumsum(counts21) - counts21
    rank = jnp.arange(S1, dtype=i32) - starts21[seg21].astype(i32)
    valid = rank < PAD
    dd = jnp.where(valid, seg21 * PAD + rank, S2 * PAD).astype(i32)

    dmin = jnp.where(valid, dd, S2 * PAD)
    suffmin = lax.cummin(dmin[::-1])[::-1]
    zone_first = suffmin[:: SEG_PER_W]
    zstart = zone_first.at[0].set(0).astype(i32)
    zend = jnp.concatenate([zstart[1:], jnp.array([S2 * PAD], i32)])

    st32 = starts10.reshape(32, SEG_PER_W)
    ct32 = counts10.reshape(32, SEG_PER_W)
    dd32 = dd.reshape(32, SEG_PER_W)
    meta_a = jnp.concatenate([st32, ct32], axis=1)
    meta_a = jnp.broadcast_to(
        meta_a.astype(f32)[:, :, None], (32, 32, 16)).reshape(-1)
    meta_b = jnp.concatenate(
        [st32, ct32, dd32, zstart[:, None], zend[:, None],
         jnp.zeros((32, 14), i32)], axis=1)
    meta_b = jnp.broadcast_to(
        meta_b.astype(f32)[:, :, None], (32, 64, 16)).reshape(-1)

    w1t = W1.T
    w2t = W2.T
    w3t = W3.T
    w3a = w3t[:256]
    w3b = w3t[256:].astype(jnp.bfloat16)
    w4t = W4.T.astype(jnp.bfloat16)
    wp1t = Wp1.T
    wp2t = Wp2.T
    b1r = b1.reshape(1, -1)
    b2r = b2.reshape(1, -1)
    b3r = b3.reshape(1, -1)
    b4r = b4.reshape(1, -1)
    bp1r = bp1.reshape(1, -1)
    bp2r = bp2.reshape(1, -1)

    full = lambda shape: pl.BlockSpec(shape, lambda t: (0,) * len(shape))
    row_blk = lambda c: pl.BlockSpec((TILE, c), lambda t: (t, 0))

    h, pos = pl.pallas_call(
        _k1_body,
        grid=(NT,),
        in_specs=[row_blk(6), full((S1, 3)), full((6, 128)), full((1, 128)),
                  full((128, 256)), full((1, 256)), full((3, 128)),
                  full((1, 128)), full((128, EMB)), full((1, EMB))],
        out_specs=[row_blk(256), full((S1, EMB))],
        out_shape=[jax.ShapeDtypeStruct((N, 256), f32),
                   jax.ShapeDtypeStruct((S1, EMB), f32)],
    )(full_features, sp_coords, w1t, b1r, w2t, b2r, wp1t, bp1r, wp2t, bp2r)

    mesh = plsc.VectorSubcoreMesh(core_axis_name="c", subcore_axis_name="s")

    sca = functools.partial(
        pl.kernel, mesh=mesh,
        out_type=jax.ShapeDtypeStruct((S1 * 256,), f32),
        scratch_types=[
            pltpu.VMEM((32 * 16,), f32),
            pltpu.VMEM((CH, 256), f32),
            pltpu.VMEM((CH, 256), f32),
            pltpu.VMEM((CH, 256), f32),
            pltpu.VMEM((SEG_PER_W * 256,), f32),
            pltpu.SemaphoreType.DMA,
            pltpu.SemaphoreType.DMA,
        ],
    )(_sca_body)
    G = sca(h, meta_a).reshape(S1, 256)

    f2 = pl.pallas_call(
        _k3_body,
        grid=(NT,),
        in_specs=[row_blk(256), row_blk(1), full((S1, 256)),
                  full((256, 512)), full((1, 512)), full((256, 512)),
                  full((512, EMB)), full((1, EMB))],
        out_specs=row_blk(EMB),
        out_shape=jax.ShapeDtypeStruct((N, EMB), f32),
        scratch_shapes=[pltpu.VMEM((S1, 512), jnp.bfloat16)],
    )(h, seg_col, G, w3a, b3r, w3b, w4t, b4r)

    scb = functools.partial(
        pl.kernel, mesh=mesh,
        out_type=[jax.ShapeDtypeStruct((OUT_ROWS * EMB,), f32),
                  jax.ShapeDtypeStruct((OUT_ROWS * EMB,), f32)],
        scratch_types=[
            pltpu.VMEM((64 * 16,), f32),
            pltpu.VMEM((CH, EMB), f32),
            pltpu.VMEM((CH, EMB), f32),
            pltpu.VMEM((CH, EMB), f32),
            pltpu.VMEM((SEG_PER_W * EMB,), f32),
            pltpu.VMEM((SEG_PER_W * EMB,), f32),
            pltpu.VMEM((EMB,), f32),
            pltpu.SemaphoreType.DMA,
            pltpu.SemaphoreType.DMA,
            pltpu.SemaphoreType.DMA,
            pltpu.SemaphoreType.DMA,
        ],
    )(_scb_body)
    tok_flat, pos_flat = scb(f2, pos.reshape(-1), meta_b)

    tok_p = tok_flat[:S2 * PAD * EMB].reshape(1, S2, PAD, EMB)
    pos_p = pos_flat[:S2 * PAD * EMB].reshape(1, S2, PAD, EMB)
    return (tok_p, pos_p)
